# Initial kernel scaffold; baseline (speedup 1.0000x reference)
import jax
import jax.numpy as jnp
from jax.experimental import pallas as pl


def kernel(input, feat, kernel_w, kernel_b, feat_w, feat_b):
    raise NotImplementedError("write your pallas kernel here")


# speedup vs baseline: 1.5396x; 1.5396x over previous
import jax
import jax.numpy as jnp
from jax import lax
from jax.experimental import pallas as pl
from jax.experimental.pallas import tpu as pltpu

_T = 1024
_NC = 3
_K = _NC * _T
_D = 10000
_BD = 400
_G = _D // _BD

_IDX = (558, 582, 554, 552, 93, 555, 580, 571, 574, 578, 566, 287, 556, 550,
        14, 551, 64, 581)


def _body(fvals_ref, accel_ref, w_ref, kb_ref, fw_ref, fb_ref, out_ref):
    proj = lax.dot_general(
        accel_ref[...].astype(jnp.bfloat16), w_ref[...].astype(jnp.bfloat16),
        (((1,), (1,)), ((), ())),
        preferred_element_type=jnp.float32)
    sample_hv = jnp.cos(proj + kb_ref[0]) * jnp.sin(proj)

    def g(i):
        p = fvals_ref[i] * fw_ref[0, i:i + 1, :]
        return jnp.cos(p + fb_ref[0, i:i + 1, :]) * jnp.sin(p)

    feat_hv = ((g(14) + g(11)) * g(16)
               * (g(4) + g(8) + g(6) + g(1) + g(5) + g(12) + g(17))
               * g(13) * (g(15) + g(2)) * g(3)
               * g(0) * g(10) * g(7) * g(9))
    out_ref[0] = jnp.where(sample_hv + feat_hv > 0, 1.0, -1.0)


def kernel(input, feat, kernel_w, kernel_b, feat_w, feat_b):
    accel = input[:, 1:4].T.reshape(1, _K)
    fvals = feat[jnp.array(_IDX, dtype=jnp.int32)]
    kb = kernel_b.reshape(_G, 1, _BD)
    fw = feat_w.reshape(18, _G, _BD).transpose(1, 0, 2)
    fb = feat_b.reshape(18, _G, _BD).transpose(1, 0, 2)
    out = pl.pallas_call(
        _body,
        grid=(_G,),
        in_specs=[
            pl.BlockSpec(memory_space=pltpu.SMEM),
            pl.BlockSpec((1, _K), lambda i: (0, 0)),
            pl.BlockSpec((_BD, _K), lambda i: (i, 0)),
            pl.BlockSpec((1, 1, _BD), lambda i: (i, 0, 0)),
            pl.BlockSpec((1, 18, _BD), lambda i: (i, 0, 0)),
            pl.BlockSpec((1, 18, _BD), lambda i: (i, 0, 0)),
        ],
        out_specs=pl.BlockSpec((1, 1, _BD), lambda i: (i, 0, 0)),
        out_shape=jax.ShapeDtypeStruct((_G, 1, _BD), jnp.float32),
        compiler_params=pltpu.CompilerParams(
            dimension_semantics=("arbitrary",)),
    )(fvals, accel, kernel_w, kb, fw, fb)
    return out.reshape(_D)


# speedup vs baseline: 1.7126x; 1.1124x over previous
import jax
import jax.numpy as jnp
from jax import lax
from jax.experimental import pallas as pl
from jax.experimental.pallas import tpu as pltpu

_T = 1024
_NC = 3
_K = _NC * _T
_D = 10000
_BD = 2000
_G = _D // _BD

_IDX = (558, 582, 554, 552, 93, 555, 580, 571, 574, 578, 566, 287, 556, 550,
        14, 551, 64, 581)


def _body(fvals_ref, accel_ref, w_ref, kb_ref, fw_ref, fb_ref, out_ref):
    proj = lax.dot_general(
        accel_ref[...].astype(jnp.bfloat16), w_ref[...].astype(jnp.bfloat16),
        (((1,), (1,)), ((), ())),
        preferred_element_type=jnp.float32)
    sample_hv = jnp.cos(proj + kb_ref[0]) * jnp.sin(proj)

    def g(i):
        p = fvals_ref[i] * fw_ref[0, i:i + 1, :]
        return jnp.cos(p + fb_ref[0, i:i + 1, :]) * jnp.sin(p)

    feat_hv = ((g(14) + g(11)) * g(16)
               * (g(4) + g(8) + g(6) + g(1) + g(5) + g(12) + g(17))
               * g(13) * (g(15) + g(2)) * g(3)
               * g(0) * g(10) * g(7) * g(9))
    out_ref[0] = jnp.where(sample_hv + feat_hv > 0, 1.0, -1.0)


def kernel(input, feat, kernel_w, kernel_b, feat_w, feat_b):
    accel = input[:, 1:4].T.reshape(1, _K)
    fvals = feat[jnp.array(_IDX, dtype=jnp.int32)]
    kb = kernel_b.reshape(_G, 1, _BD)
    fw = feat_w.reshape(18, _G, _BD).transpose(1, 0, 2)
    fb = feat_b.reshape(18, _G, _BD).transpose(1, 0, 2)
    out = pl.pallas_call(
        _body,
        grid=(_G,),
        in_specs=[
            pl.BlockSpec(memory_space=pltpu.SMEM),
            pl.BlockSpec((1, _K), lambda i: (0, 0)),
            pl.BlockSpec((_BD, _K), lambda i: (i, 0)),
            pl.BlockSpec((1, 1, _BD), lambda i: (i, 0, 0)),
            pl.BlockSpec((1, 18, _BD), lambda i: (i, 0, 0)),
            pl.BlockSpec((1, 18, _BD), lambda i: (i, 0, 0)),
        ],
        out_specs=pl.BlockSpec((1, 1, _BD), lambda i: (i, 0, 0)),
        out_shape=jax.ShapeDtypeStruct((_G, 1, _BD), jnp.float32),
        compiler_params=pltpu.CompilerParams(
            dimension_semantics=("arbitrary",)),
    )(fvals, accel, kernel_w, kb, fw, fb)
    return out.reshape(_D)


# speedup vs baseline: 1.8535x; 1.0823x over previous
import jax
import jax.numpy as jnp
from jax import lax
from jax.experimental import pallas as pl
from jax.experimental.pallas import tpu as pltpu

_T = 1024
_NC = 3
_K = _NC * _T
_D = 10000
_BD = 1000
_G = _D // _BD

_IDX = (558, 582, 554, 552, 93, 555, 580, 571, 574, 578, 566, 287, 556, 550,
        14, 551, 64, 581)


def _body(fvals_ref, accel_ref, w_ref, kb_ref, fw_ref, fb_ref, out_ref):
    proj = lax.dot_general(
        accel_ref[...].astype(jnp.bfloat16), w_ref[...].astype(jnp.bfloat16),
        (((1,), (1,)), ((), ())),
        preferred_element_type=jnp.float32)
    sample_hv = jnp.cos(proj + kb_ref[0]) * jnp.sin(proj)

    def g(i):
        p = fvals_ref[i] * fw_ref[0, i:i + 1, :]
        return jnp.cos(p + fb_ref[0, i:i + 1, :]) * jnp.sin(p)

    feat_hv = ((g(14) + g(11)) * g(16)
               * (g(4) + g(8) + g(6) + g(1) + g(5) + g(12) + g(17))
               * g(13) * (g(15) + g(2)) * g(3)
               * g(0) * g(10) * g(7) * g(9))
    out_ref[0] = jnp.where(sample_hv + feat_hv > 0, 1.0, -1.0)


def kernel(input, feat, kernel_w, kernel_b, feat_w, feat_b):
    accel = input[:, 1:4].T.reshape(1, _K)
    fvals = feat[jnp.array(_IDX, dtype=jnp.int32)]
    kb = kernel_b.reshape(_G, 1, _BD)
    fw = feat_w.reshape(18, _G, _BD).transpose(1, 0, 2)
    fb = feat_b.reshape(18, _G, _BD).transpose(1, 0, 2)
    out = pl.pallas_call(
        _body,
        grid=(_G,),
        in_specs=[
            pl.BlockSpec(memory_space=pltpu.SMEM),
            pl.BlockSpec((1, _K), lambda i: (0, 0)),
            pl.BlockSpec((_BD, _K), lambda i: (i, 0)),
            pl.BlockSpec((1, 1, _BD), lambda i: (i, 0, 0)),
            pl.BlockSpec((1, 18, _BD), lambda i: (i, 0, 0)),
            pl.BlockSpec((1, 18, _BD), lambda i: (i, 0, 0)),
        ],
        out_specs=pl.BlockSpec((1, 1, _BD), lambda i: (i, 0, 0)),
        out_shape=jax.ShapeDtypeStruct((_G, 1, _BD), jnp.float32),
        compiler_params=pltpu.CompilerParams(
            dimension_semantics=("arbitrary",)),
    )(fvals, accel, kernel_w, kb, fw, fb)
    return out.reshape(_D)
